# Initial kernel scaffold; baseline (speedup 1.0000x reference)
import jax
import jax.numpy as jnp
from jax.experimental import pallas as pl


def kernel(x, weight):
    raise NotImplementedError("write your pallas kernel here")


# speedup vs baseline: 3.2803x; 3.2803x over previous
import jax
import jax.numpy as jnp
from jax.experimental import pallas as pl
from jax.experimental.pallas import tpu as pltpu
from jax.experimental.pallas import tpu_sc as plsc

_WINDOW = 256


def kernel(x, weight):
    b, h = x.shape
    n = b * h
    dim = weight.shape[1]
    idx = x.reshape(1, n)

    mesh = plsc.VectorSubcoreMesh(core_axis_name="c", subcore_axis_name="s")

    @pl.kernel(
        out_type=jax.ShapeDtypeStruct((n, dim), weight.dtype),
        mesh=mesh,
    )
    def gather_kernel(w_hbm, i_hbm, o_hbm):
        def body(i_vmem, o_vmem):
            pltpu.sync_copy(w_hbm.at[i_vmem.at[0]], o_vmem)

        pltpu.emit_pipeline(
            body,
            grid=(n // _WINDOW,),
            in_specs=[pl.BlockSpec((1, _WINDOW), index_map=lambda i: (0, i))],
            out_specs=[pl.BlockSpec((_WINDOW, dim), index_map=lambda i: (i, 0))],
            core_axis_name=("c", "s"),
            dimension_semantics=(pltpu.PARALLEL,),
        )(i_hbm, o_hbm)

    out = gather_kernel(weight, idx)
    return out.reshape(b, h, dim)


# speedup vs baseline: 5.4894x; 1.6735x over previous
import jax
import jax.numpy as jnp
from jax import lax
from jax.experimental import pallas as pl
from jax.experimental.pallas import tpu as pltpu
from jax.experimental.pallas import tpu_sc as plsc

_NUM_CORES = 2
_NUM_SUBCORES = 16
_NUM_WORKERS = _NUM_CORES * _NUM_SUBCORES
_ROWS_PER_CHUNK = 8


def kernel(x, weight):
    b, h = x.shape
    n = b * h
    dim = weight.shape[1]
    rows_per_w = b // _NUM_WORKERS
    idx_per_w = rows_per_w * h
    chunk_idx = _ROWS_PER_CHUNK * h
    idx = x.reshape(n)

    mesh = plsc.VectorSubcoreMesh(core_axis_name="c", subcore_axis_name="s")

    @pl.kernel(
        out_type=jax.ShapeDtypeStruct((b, h, dim), weight.dtype),
        mesh=mesh,
        scratch_types=[
            pltpu.VMEM((idx_per_w,), jnp.int32),
            pltpu.VMEM((chunk_idx, dim), jnp.float32),
            pltpu.SemaphoreType.DMA,
        ],
    )
    def gather_kernel(w_hbm, i_hbm, o_hbm, idx_v, rows_v, sem):
        wid = lax.axis_index("s") * _NUM_CORES + lax.axis_index("c")
        row_base = wid * rows_per_w
        pltpu.sync_copy(i_hbm.at[pl.ds(wid * idx_per_w, idx_per_w)], idx_v)

        @pl.loop(0, rows_per_w, step=_ROWS_PER_CHUNK)
        def _(r):
            pltpu.async_copy(
                w_hbm.at[idx_v.at[pl.ds(r * h, chunk_idx)]], rows_v, sem
            ).wait()
            for j in range(_ROWS_PER_CHUNK):
                pltpu.sync_copy(
                    rows_v.at[pl.ds(j * h, h)], o_hbm.at[row_base + r + j]
                )

    out = gather_kernel(weight, idx)
    return out


# speedup vs baseline: 5.9099x; 1.0766x over previous
import jax
import jax.numpy as jnp
from jax import lax
from jax.experimental import pallas as pl
from jax.experimental.pallas import tpu as pltpu
from jax.experimental.pallas import tpu_sc as plsc

_NUM_CORES = 2
_NUM_SUBCORES = 16
_NUM_WORKERS = _NUM_CORES * _NUM_SUBCORES
_ROWS_PER_CHUNK = 8


def kernel(x, weight):
    b, h = x.shape
    n = b * h
    dim = weight.shape[1]
    rows_per_w = b // _NUM_WORKERS
    idx_per_w = rows_per_w * h
    chunk_idx = _ROWS_PER_CHUNK * h
    idx = x.reshape(n)

    mesh = plsc.VectorSubcoreMesh(core_axis_name="c", subcore_axis_name="s")

    @pl.kernel(
        out_type=jax.ShapeDtypeStruct((b, h, dim), weight.dtype),
        mesh=mesh,
        scratch_types=[
            pltpu.VMEM((idx_per_w,), jnp.int32),
            pltpu.VMEM((chunk_idx, dim), jnp.float32),
            pltpu.VMEM((chunk_idx, dim), jnp.float32),
            pltpu.SemaphoreType.DMA,
            pltpu.SemaphoreType.DMA,
        ],
    )
    def gather_kernel(w_hbm, i_hbm, o_hbm, idx_v, rows_v0, rows_v1, sem0, sem1):
        wid = lax.axis_index("s") * _NUM_CORES + lax.axis_index("c")
        row_base = wid * rows_per_w
        n_chunks = rows_per_w // _ROWS_PER_CHUNK
        pltpu.sync_copy(i_hbm.at[pl.ds(wid * idx_per_w, idx_per_w)], idx_v)

        def gather_start(c, buf, sem):
            pltpu.async_copy(
                w_hbm.at[idx_v.at[pl.ds(c * chunk_idx, chunk_idx)]], buf, sem
            )

        def gather_wait(c, buf, sem):
            pltpu.make_async_copy(
                w_hbm.at[idx_v.at[pl.ds(c * chunk_idx, chunk_idx)]], buf, sem
            ).wait()

        def writeback(c, buf):
            for j in range(_ROWS_PER_CHUNK):
                pltpu.sync_copy(
                    buf.at[pl.ds(j * h, h)],
                    o_hbm.at[row_base + c * _ROWS_PER_CHUNK + j],
                )

        gather_start(0, rows_v0, sem0)

        @pl.loop(0, n_chunks, step=2)
        def _(c):
            gather_wait(c, rows_v0, sem0)
            gather_start(c + 1, rows_v1, sem1)
            writeback(c, rows_v0)

            @pl.when(c + 2 < n_chunks)
            def _():
                gather_start(c + 2, rows_v0, sem0)

            gather_wait(c + 1, rows_v1, sem1)
            writeback(c + 1, rows_v1)

    out = gather_kernel(weight, idx)
    return out


# speedup vs baseline: 5.9185x; 1.0015x over previous
import jax
import jax.numpy as jnp
from jax import lax
from jax.experimental import pallas as pl
from jax.experimental.pallas import tpu as pltpu
from jax.experimental.pallas import tpu_sc as plsc

_NUM_CORES = 2
_NUM_SUBCORES = 16
_NUM_WORKERS = _NUM_CORES * _NUM_SUBCORES
_ROWS_PER_CHUNK = 8


def kernel(x, weight):
    b, h = x.shape
    n = b * h
    dim = weight.shape[1]
    rows_per_w = b // _NUM_WORKERS
    idx_per_w = rows_per_w * h
    chunk_idx = _ROWS_PER_CHUNK * h
    idx = x.reshape(n)

    mesh = plsc.VectorSubcoreMesh(core_axis_name="c", subcore_axis_name="s")

    @pl.kernel(
        out_type=jax.ShapeDtypeStruct((b, h, dim), weight.dtype),
        mesh=mesh,
        scratch_types=[
            pltpu.VMEM((idx_per_w,), jnp.int32),
            pltpu.VMEM((chunk_idx, dim), jnp.float32),
            pltpu.VMEM((chunk_idx, dim), jnp.float32),
            pltpu.SemaphoreType.DMA,
            pltpu.SemaphoreType.DMA,
            pltpu.SemaphoreType.DMA,
            pltpu.SemaphoreType.DMA,
        ],
    )
    def gather_kernel(
        w_hbm, i_hbm, o_hbm, idx_v, rows_v0, rows_v1, sem0, sem1, semw0, semw1
    ):
        wid = lax.axis_index("s") * _NUM_CORES + lax.axis_index("c")
        row_base = wid * rows_per_w
        n_chunks = rows_per_w // _ROWS_PER_CHUNK
        pltpu.sync_copy(i_hbm.at[pl.ds(wid * idx_per_w, idx_per_w)], idx_v)

        def gather_start(c, buf, sem):
            pltpu.async_copy(
                w_hbm.at[idx_v.at[pl.ds(c * chunk_idx, chunk_idx)]], buf, sem
            )

        def gather_wait(c, buf, sem):
            pltpu.make_async_copy(
                w_hbm.at[idx_v.at[pl.ds(c * chunk_idx, chunk_idx)]], buf, sem
            ).wait()

        def wb_start(c, buf, sem):
            for j in range(_ROWS_PER_CHUNK):
                pltpu.async_copy(
                    buf.at[pl.ds(j * h, h)],
                    o_hbm.at[row_base + c * _ROWS_PER_CHUNK + j],
                    sem,
                )

        def wb_drain(c, buf, sem):
            for j in range(_ROWS_PER_CHUNK):
                pltpu.make_async_copy(
                    buf.at[pl.ds(j * h, h)],
                    o_hbm.at[row_base + c * _ROWS_PER_CHUNK + j],
                    sem,
                ).wait()

        gather_start(0, rows_v0, sem0)
        gather_start(1, rows_v1, sem1)

        @pl.loop(0, n_chunks, step=2)
        def _(c):
            gather_wait(c, rows_v0, sem0)
            wb_start(c, rows_v0, semw0)
            wb_drain(c, rows_v0, semw0)

            @pl.when(c + 2 < n_chunks)
            def _():
                gather_start(c + 2, rows_v0, sem0)

            gather_wait(c + 1, rows_v1, sem1)
            wb_start(c + 1, rows_v1, semw1)
            wb_drain(c + 1, rows_v1, semw1)

            @pl.when(c + 3 < n_chunks)
            def _():
                gather_start(c + 3, rows_v1, sem1)

    out = gather_kernel(weight, idx)
    return out


# speedup vs baseline: 10.4294x; 1.7622x over previous
import jax
import jax.numpy as jnp
from jax import lax
from jax.experimental import pallas as pl
from jax.experimental.pallas import tpu as pltpu
from jax.experimental.pallas import tpu_sc as plsc

_NUM_CORES = 2
_NUM_SUBCORES = 16
_NUM_WORKERS = _NUM_CORES * _NUM_SUBCORES
_CHUNK = 400


def kernel(x, weight):
    b, h = x.shape
    n = b * h
    dim = weight.shape[1]
    idx_per_w = n // _NUM_WORKERS
    n_chunks = idx_per_w // _CHUNK
    idx = x.T.reshape(n)

    mesh = plsc.VectorSubcoreMesh(core_axis_name="c", subcore_axis_name="s")

    @pl.kernel(
        out_type=jax.ShapeDtypeStruct((n, dim), weight.dtype),
        mesh=mesh,
        scratch_types=[
            pltpu.VMEM((idx_per_w,), jnp.int32),
            pltpu.VMEM((_CHUNK, dim), jnp.float32),
            pltpu.VMEM((_CHUNK, dim), jnp.float32),
            pltpu.SemaphoreType.DMA,
            pltpu.SemaphoreType.DMA,
            pltpu.SemaphoreType.DMA,
            pltpu.SemaphoreType.DMA,
        ],
    )
    def gather_kernel(
        w_hbm, i_hbm, o_hbm, idx_v, rows_v0, rows_v1, sem0, sem1, semw0, semw1
    ):
        wid = lax.axis_index("s") * _NUM_CORES + lax.axis_index("c")
        base = wid * idx_per_w
        pltpu.sync_copy(i_hbm.at[pl.ds(base, idx_per_w)], idx_v)

        def gather_start(c, buf, sem):
            pltpu.async_copy(
                w_hbm.at[idx_v.at[pl.ds(c * _CHUNK, _CHUNK)]], buf, sem
            )

        def gather_wait(c, buf, sem):
            pltpu.make_async_copy(
                w_hbm.at[idx_v.at[pl.ds(c * _CHUNK, _CHUNK)]], buf, sem
            ).wait()

        def wb_start(c, buf, sem):
            pltpu.async_copy(buf, o_hbm.at[pl.ds(base + c * _CHUNK, _CHUNK)], sem)

        def wb_drain(c, buf, sem):
            pltpu.make_async_copy(
                buf, o_hbm.at[pl.ds(base + c * _CHUNK, _CHUNK)], sem
            ).wait()

        gather_start(0, rows_v0, sem0)
        gather_start(1, rows_v1, sem1)

        @pl.loop(0, n_chunks, step=2)
        def _(c):
            gather_wait(c, rows_v0, sem0)
            wb_start(c, rows_v0, semw0)
            wb_drain(c, rows_v0, semw0)

            @pl.when(c + 2 < n_chunks)
            def _():
                gather_start(c + 2, rows_v0, sem0)

            gather_wait(c + 1, rows_v1, sem1)
            wb_start(c + 1, rows_v1, semw1)
            wb_drain(c + 1, rows_v1, semw1)

            @pl.when(c + 3 < n_chunks)
            def _():
                gather_start(c + 3, rows_v1, sem1)

    out = gather_kernel(weight, idx)
    return out.reshape(h, b, dim).transpose(1, 0, 2)
